# Initial kernel scaffold; baseline (speedup 1.0000x reference)
import jax
import jax.numpy as jnp
from jax.experimental import pallas as pl


def kernel(inputs, embedding):
    raise NotImplementedError("write your pallas kernel here")


# speedup vs baseline: 1.0023x; 1.0023x over previous
import functools

import jax
import jax.numpy as jnp
from jax import lax
from jax.experimental import pallas as pl
from jax.experimental.pallas import tpu as pltpu
from jax.experimental.pallas import tpu_sc as plsc

_D = 32
_N = 16384
_K = 8192
_M = 512
_NB = _N // _M

_NC = 2
_NS = 16
_NW = _NC * _NS
_BPW = _N // _NW
_CH = _BPW // 128


def _argmin_codes(x, emb, xsq, esq):
    def body(x_ref, e_ref, xsq_ref, esq_ref, o_ref):
        xb = x_ref[...]
        e = e_ref[...]
        dot_t = lax.dot_general(
            e, xb, (((0,), (1,)), ((), ())),
            preferred_element_type=jnp.float32,
        )
        dist_t = (xsq_ref[0] - 2.0 * dot_t) + esq_ref[...]
        minv = jnp.min(dist_t, axis=0, keepdims=True)
        ii = lax.broadcasted_iota(jnp.int32, dist_t.shape, 0)
        idx = jnp.min(jnp.where(dist_t == minv, ii, jnp.int32(_K)), axis=0)
        o_ref[...] = idx.reshape(1, 1, _M)

    return pl.pallas_call(
        body,
        grid=(_NB,),
        in_specs=[
            pl.BlockSpec((_M, _D), lambda i: (i, 0)),
            pl.BlockSpec((_D, _K), lambda i: (0, 0)),
            pl.BlockSpec((1, 1, _M), lambda i: (i, 0, 0)),
            pl.BlockSpec((_K, 1), lambda i: (0, 0)),
        ],
        out_specs=pl.BlockSpec((1, 1, _M), lambda i: (i, 0, 0)),
        out_shape=jax.ShapeDtypeStruct((_NB, 1, _M), jnp.int32),
    )(x, emb, xsq, esq)


def _gather_codes(table, idx):
    mesh = plsc.VectorSubcoreMesh(core_axis_name="c", subcore_axis_name="s")

    @functools.partial(
        pl.kernel,
        mesh=mesh,
        compiler_params=pltpu.CompilerParams(use_tc_tiling_on_sc=False),
        out_type=jax.ShapeDtypeStruct((_NW, _CH, 128, _D), jnp.float32),
        scratch_types=[
            pltpu.VMEM((_CH, 128), jnp.int32),
            pltpu.VMEM((_CH, 128, _D), jnp.float32),
            pltpu.SemaphoreType.DMA,
        ],
    )
    def k(table_hbm, idx_hbm, out_hbm, idx_v, rows_v, sem):
        wid = lax.axis_index("s") * _NC + lax.axis_index("c")
        pltpu.sync_copy(idx_hbm.at[wid], idx_v)
        copies = [
            pltpu.async_copy(table_hbm.at[idx_v.at[j]], rows_v.at[j], sem)
            for j in range(_CH)
        ]
        for c in copies:
            c.wait()
        pltpu.sync_copy(rows_v, out_hbm.at[wid])

    return k(table, idx)


def kernel(inputs, embedding):
    emb_dim = embedding.shape[0]
    x = inputs.reshape(-1, emb_dim)
    distance = (
        jnp.sum(jnp.power(x, 2), axis=-1, keepdims=True)
        - 2.0 * jnp.dot(x, embedding)
        + jnp.sum(jnp.power(embedding, 2), axis=0, keepdims=True)
    )
    arg_distance = jnp.argmax(-distance, axis=1).reshape(inputs.shape[:-1])
    idx = arg_distance.reshape(_NW, _CH, 128).astype(jnp.int32)
    out = _gather_codes(embedding.T, idx)
    return out.reshape(inputs.shape)


# speedup vs baseline: 1.0807x; 1.0782x over previous
import functools

import jax
import jax.numpy as jnp
from jax import lax
from jax.experimental import pallas as pl
from jax.experimental.pallas import tpu as pltpu
from jax.experimental.pallas import tpu_sc as plsc

_D = 32
_N = 16384
_K = 8192
_M = 512
_NB = _N // _M
_SPAN = 4096
_NS_SPANS = _K // _SPAN

_NC = 2
_NSUB = 16
_NW = _NC * _NSUB
_BPW = _N // _NW
_CH = _BPW // 128


def _argmin_codes(x, emb, xsq, esq):
    def body(x_ref, e_ref, xsq_ref, esq_ref, o_ref):
        xb = x_ref[...]
        e = e_ref[...]
        dot_t = lax.dot_general(
            e, xb, (((0,), (1,)), ((), ())),
            preferred_element_type=jnp.float32,
        )
        dist_t = (xsq_ref[0] - 2.0 * dot_t) + esq_ref[...]

        big = jnp.int32(_K)
        acc_v = None
        acc_i = None
        for b in range(_NS_SPANS):
            sub = dist_t[b * _SPAN:(b + 1) * _SPAN, :]
            minv = jnp.min(sub, axis=0)
            ii = jnp.int32(b * _SPAN) + lax.broadcasted_iota(
                jnp.int32, (_SPAN, _M), 0)
            idx = jnp.min(jnp.where(sub == minv[None, :], ii, big), axis=0)
            wv = minv.astype(jnp.bfloat16).astype(jnp.float32)
            if acc_v is None:
                acc_v, acc_i = wv, idx
            else:
                m = minv < acc_v
                acc_v = jnp.where(m, wv, acc_v)
                acc_i = jnp.where(m, idx, acc_i)
        o_ref[...] = acc_i.reshape(1, 1, _M)

    return pl.pallas_call(
        body,
        grid=(_NB,),
        in_specs=[
            pl.BlockSpec((_M, _D), lambda i: (i, 0)),
            pl.BlockSpec((_D, _K), lambda i: (0, 0)),
            pl.BlockSpec((1, 1, _M), lambda i: (i, 0, 0)),
            pl.BlockSpec((_K, 1), lambda i: (0, 0)),
        ],
        out_specs=pl.BlockSpec((1, 1, _M), lambda i: (i, 0, 0)),
        out_shape=jax.ShapeDtypeStruct((_NB, 1, _M), jnp.int32),
    )(x, emb, xsq, esq)


def _gather_codes(table, idx):
    mesh = plsc.VectorSubcoreMesh(core_axis_name="c", subcore_axis_name="s")

    @functools.partial(
        pl.kernel,
        mesh=mesh,
        compiler_params=pltpu.CompilerParams(use_tc_tiling_on_sc=False),
        out_type=jax.ShapeDtypeStruct((_NW, _CH, 128, _D), jnp.float32),
        scratch_types=[
            pltpu.VMEM((_CH, 128), jnp.int32),
            pltpu.VMEM((_CH, 128, _D), jnp.float32),
            pltpu.SemaphoreType.DMA,
        ],
    )
    def k(table_hbm, idx_hbm, out_hbm, idx_v, rows_v, sem):
        wid = lax.axis_index("s") * _NC + lax.axis_index("c")
        pltpu.sync_copy(idx_hbm.at[wid], idx_v)
        copies = [
            pltpu.async_copy(table_hbm.at[idx_v.at[j]], rows_v.at[j], sem)
            for j in range(_CH)
        ]
        for c in copies:
            c.wait()
        pltpu.sync_copy(rows_v, out_hbm.at[wid])

    return k(table, idx)


def kernel(inputs, embedding):
    xsq = jnp.sum(jnp.power(inputs, 2), axis=-1).reshape(_NB, 1, _M)
    esq = jnp.sum(jnp.power(embedding, 2), axis=0).reshape(_K, 1)
    x = inputs.reshape(-1, embedding.shape[0]).astype(jnp.bfloat16)
    idx = _argmin_codes(x, embedding, xsq, esq).reshape(_NW, _CH, 128)
    out = _gather_codes(embedding.T, idx)
    return out.reshape(inputs.shape)
